# baseline (speedup 1.0000x reference)
import functools

import jax
import jax.numpy as jnp
from jax import lax
from jax.experimental import pallas as pl
from jax.experimental.pallas import tpu as pltpu
from jax.experimental.pallas import tpu_sc as plsc

N = 10000
E = 320000
D = 128
H = 4
DA = 144
NS = 16
L = 16
EPT = E // NS
CH = 80
NCHUNK = EPT // CH
NPAD = 10240
RPT = NPAD // NS
RPT_LAST = N - RPT * (NS - 1)
ALPHA = 0.2
ACT_SLOPE = 0.01
MIN_NORM = 1e-15
PROJ_EPS = 4e-3
BN = 2000


def _artanh(v):
    v = jnp.clip(v, -1.0 + 1e-7, 1.0 - 1e-7)
    return 0.5 * (jnp.log1p(v) - jnp.log1p(-v))


def _tc_prep_body(x_ref, w_ref, a_ref, h_ref, s_ref):
    xb = x_ref[...]
    nrm = jnp.sqrt(jnp.sum(xb * xb, axis=-1, keepdims=True))
    nrm = jnp.maximum(nrm, MIN_NORM)
    xt = xb * (_artanh(nrm) / nrm)
    one = jnp.ones((xb.shape[0], 1), jnp.float32)
    pad = jnp.zeros((xb.shape[0], DA - D - 1), jnp.float32)
    s1s, s2s = [], []
    for i in range(H):
        h = jnp.dot(xt, w_ref[i], preferred_element_type=jnp.float32)
        h_ref[i] = jnp.concatenate([h, one, pad], axis=-1)
        s1s.append(jnp.sum(h * a_ref[i, :D][None, :], axis=-1, keepdims=True))
        s2s.append(jnp.sum(h * a_ref[i, D:][None, :], axis=-1, keepdims=True))
    s_ref[...] = jnp.concatenate(s1s + s2s, axis=-1)


def _tc_prep(x, W, a):
    return pl.pallas_call(
        _tc_prep_body,
        grid=(N // BN,),
        in_specs=[
            pl.BlockSpec((BN, D), lambda i: (i, 0)),
            pl.BlockSpec((H, D, D), lambda i: (0, 0, 0)),
            pl.BlockSpec((H, 2 * D), lambda i: (0, 0)),
        ],
        out_specs=[
            pl.BlockSpec((H, BN, DA), lambda i: (0, i, 0)),
            pl.BlockSpec((BN, 2 * H), lambda i: (i, 0)),
        ],
        out_shape=[
            jax.ShapeDtypeStruct((H, N, DA), jnp.float32),
            jax.ShapeDtypeStruct((N, 2 * H), jnp.float32),
        ],
    )(x, W, a)


_sc_mesh = plsc.VectorSubcoreMesh(core_axis_name="c", subcore_axis_name="s")


NB = 3
_SC_PARAMS = pltpu.CompilerParams(needs_layout_passes=False,
                                  use_tc_tiling_on_sc=False)


@functools.partial(
    pl.kernel,
    out_type=jax.ShapeDtypeStruct((H * E,), jnp.float32),
    mesh=_sc_mesh,
    compiler_params=_SC_PARAMS,
    scratch_types=[
        pltpu.VMEM((EPT,), jnp.int32),
        pltpu.VMEM((EPT,), jnp.int32),
        pltpu.VMEM((N,), jnp.float32),
        pltpu.VMEM((N,), jnp.float32),
        pltpu.VMEM((EPT,), jnp.float32),
    ],
)
def _sc_edge(s_hbm, src_hbm, dst_hbm, e_hbm,
             srcall_v, dstall_v, s1_v, s2_v, e_v):
    c = lax.axis_index("c")
    t = lax.axis_index("s")
    ebase = t * EPT
    pltpu.sync_copy(src_hbm.at[pl.ds(ebase, EPT)], srcall_v)
    pltpu.sync_copy(dst_hbm.at[pl.ds(ebase, EPT)], dstall_v)
    for p in range(2):
        hh = 2 * p + c
        pltpu.sync_copy(s_hbm.at[pl.ds(hh * N, N)], s1_v)
        pltpu.sync_copy(s_hbm.at[pl.ds((H + hh) * N, N)], s2_v)

        def grp(g, carry):
            sl = pl.ds(g * L, L)
            si = srcall_v[sl]
            di = dstall_v[sl]
            zt = plsc.load_gather(s1_v, [si]) + plsc.load_gather(s2_v, [di])
            lr = jnp.where(zt >= 0, zt, ALPHA * zt)
            e_v[sl] = jnp.exp(-lr)
            return carry

        lax.fori_loop(0, EPT // L, grp, 0)
        pltpu.sync_copy(e_v, e_hbm.at[pl.ds(hh * E + ebase, EPT)])


@functools.partial(
    pl.kernel,
    out_type=jax.ShapeDtypeStruct((H * N, DA), jnp.float32),
    mesh=_sc_mesh,
    compiler_params=_SC_PARAMS,
    scratch_types=(
        [pltpu.VMEM((CH,), jnp.int32)] * NB
        + [pltpu.VMEM((CH,), jnp.int32)] * NB
        + [pltpu.VMEM((CH,), jnp.int32)] * NB
        + [pltpu.VMEM((CH,), jnp.int32)] * NB
        + [pltpu.VMEM((CH + L,), jnp.float32)] * NB
        + [pltpu.VMEM((CH, DA), jnp.float32)] * NB
        + [pltpu.VMEM_SHARED((NPAD, DA), jnp.float32)]
        + [pltpu.SemaphoreType.DMA] * (3 * NB)
    ),
)
def _sc_agg(h_hbm, e_hbm, src_hbm, dst_hbm, z_hbm, out_hbm, *refs):
    srcv = refs[0:NB]
    dstv = refs[NB:2 * NB]
    srcsc = refs[2 * NB:3 * NB]
    dstb = refs[3 * NB:4 * NB]
    epad = refs[4 * NB:5 * NB]
    rows = refs[5 * NB:6 * NB]
    hp_sh = refs[6 * NB]
    semi = refs[6 * NB + 1:7 * NB + 1]
    semg = refs[7 * NB + 1:8 * NB + 1]
    sems = refs[8 * NB + 1:9 * NB + 1]
    c = lax.axis_index("c")
    t = lax.axis_index("s")
    rbase = t * RPT
    ebase = t * EPT
    for p in range(2):
        hh = 2 * p + c
        hoff = hh * N
        eoff = hh * E + ebase
        pltpu.sync_copy(z_hbm, hp_sh.at[pl.ds(rbase, RPT)])
        plsc.subcore_barrier()

        def start_idx(kc, cur):
            base = kc * CH
            pltpu.async_copy(src_hbm.at[pl.ds(ebase + base, CH)],
                             srcv[cur], semi[cur])
            pltpu.async_copy(dst_hbm.at[pl.ds(ebase + base, CH)],
                             dstv[cur], semi[cur])
            pltpu.async_copy(e_hbm.at[pl.ds(eoff + base, CH)],
                             epad[cur].at[pl.ds(0, CH)], semi[cur])

        def wait_idx(cur):
            pltpu.make_async_copy(src_hbm.at[pl.ds(0, CH)], srcv[cur],
                                  semi[cur]).wait()
            pltpu.make_async_copy(dst_hbm.at[pl.ds(0, CH)], dstv[cur],
                                  semi[cur]).wait()
            pltpu.make_async_copy(e_hbm.at[pl.ds(0, CH)],
                                  epad[cur].at[pl.ds(0, CH)],
                                  semi[cur]).wait()

        def compute(cur):
            for j in range(CH // L):
                sl = pl.ds(j * L, L)
                srcsc[cur][sl] = srcv[cur][sl]
                dstb[cur][sl] = dstv[cur][sl] + hoff

        def start_gather(cur):
            pltpu.async_copy(h_hbm.at[dstb[cur]], rows[cur], semg[cur])

        def wait_gather(cur):
            pltpu.make_async_copy(h_hbm.at[pl.ds(0, CH)], rows[cur],
                                  semg[cur]).wait()

        def scale(cur):
            def body(j, carry):
                jb = j * 2
                for dj in range(2):
                    idx = jnp.full((L,), jb + dj, jnp.int32)
                    ee = plsc.load_gather(epad[cur], [idx])
                    for r in range(DA // L):
                        rsl = pl.ds(r * L, L)
                        rows[cur][jb + dj, rsl] = rows[cur][jb + dj, rsl] * ee
                return carry

            lax.fori_loop(0, CH // 2, body, 0)

        def start_scatter(cur):
            pltpu.async_copy(rows[cur], hp_sh.at[srcsc[cur]], sems[cur],
                             add=True)

        def wait_scatter(cur):
            pltpu.make_async_copy(h_hbm.at[pl.ds(0, CH)], rows[cur],
                                  sems[cur]).wait()

        def section(k, cur, drain, prep, prefetch):
            nxt = (cur + 1) % NB
            if drain:
                wait_scatter(nxt)
            if prep:
                wait_idx(nxt)
                compute(nxt)
                start_gather(nxt)
            wait_gather(cur)
            scale(cur)
            start_scatter(cur)
            if prefetch:
                start_idx(k + 3, cur)

        start_idx(0, 0)
        start_idx(1, 1)
        start_idx(2, 2)
        wait_idx(0)
        compute(0)
        start_gather(0)
        section(0, 0, drain=False, prep=True, prefetch=True)
        section(1, 1, drain=False, prep=True, prefetch=True)

        def tri_body(i, carry):
            k0 = 2 + NB * i
            for b in range(NB):
                section(k0 + b, (2 + b) % NB, drain=True, prep=True,
                        prefetch=True)
            return carry

        lax.fori_loop(0, (NCHUNK - 5 - 2) // NB, tri_body, 0)
        section(NCHUNK - 5, (NCHUNK - 5) % NB, True, True, True)
        section(NCHUNK - 4, (NCHUNK - 4) % NB, True, True, True)
        section(NCHUNK - 3, (NCHUNK - 3) % NB, True, True, False)
        section(NCHUNK - 2, (NCHUNK - 2) % NB, True, True, False)
        section(NCHUNK - 1, (NCHUNK - 1) % NB, True, False, False)
        wait_scatter((NCHUNK - 2) % NB)
        wait_scatter((NCHUNK - 1) % NB)
        plsc.subcore_barrier()

        @pl.when(t < NS - 1)
        def _():
            pltpu.sync_copy(hp_sh.at[pl.ds(rbase, RPT)],
                            out_hbm.at[pl.ds(hoff + rbase, RPT)])

        @pl.when(t == NS - 1)
        def _():
            pltpu.sync_copy(hp_sh.at[pl.ds(rbase, RPT_LAST)],
                            out_hbm.at[pl.ds(hoff + rbase, RPT_LAST)])

        plsc.subcore_barrier()


def _tc_finish_body(hp_ref, out_ref):
    hp = hp_ref[...]
    num = hp[:, :, :D]
    den = hp[:, :, D:D + 1]
    g = num / den
    g = jnp.where(g >= 0, g, ACT_SLOPE * g)
    u = jnp.concatenate([g[i] for i in range(H)], axis=-1)
    un = jnp.sqrt(jnp.sum(u * u, axis=-1, keepdims=True))
    un = jnp.maximum(un, MIN_NORM)
    v = jnp.tanh(un) * u / un
    vn = jnp.sqrt(jnp.sum(v * v, axis=-1, keepdims=True))
    vn = jnp.maximum(vn, MIN_NORM)
    maxn = 1.0 - PROJ_EPS
    out_ref[...] = jnp.where(vn > maxn, v / vn * maxn, v)


def _tc_finish(hp):
    return pl.pallas_call(
        _tc_finish_body,
        grid=(N // BN,),
        in_specs=[pl.BlockSpec((H, BN, DA), lambda i: (0, i, 0))],
        out_specs=pl.BlockSpec((BN, H * D), lambda i: (i, 0)),
        out_shape=jax.ShapeDtypeStruct((N, H * D), jnp.float32),
    )(hp)


def kernel(x, adj, W, a):
    src = adj[0]
    dst = adj[1]
    h_aug, s = _tc_prep(x, W, a)
    s_flat = s.T.reshape(-1)
    h_flat = h_aug.reshape(H * N, DA)
    e = _sc_edge(s_flat, src, dst)
    zeros = jnp.zeros((RPT, DA), jnp.float32)
    hp = _sc_agg(h_flat, e, src, dst, zeros)
    return _tc_finish(hp.reshape(H, N, DA))


# speedup vs baseline: 1.0050x; 1.0050x over previous
import functools

import jax
import jax.numpy as jnp
from jax import lax
from jax.experimental import pallas as pl
from jax.experimental.pallas import tpu as pltpu
from jax.experimental.pallas import tpu_sc as plsc

N = 10000
E = 320000
D = 128
H = 4
DA = 144
NS = 16
L = 16
EPT = E // NS
CH = 80
NCHUNK = EPT // CH
NPAD = 10240
RPT = NPAD // NS
RPT_LAST = N - RPT * (NS - 1)
ALPHA = 0.2
ACT_SLOPE = 0.01
MIN_NORM = 1e-15
PROJ_EPS = 4e-3
BN = 2000


def _artanh(v):
    v = jnp.clip(v, -1.0 + 1e-7, 1.0 - 1e-7)
    return 0.5 * (jnp.log1p(v) - jnp.log1p(-v))


def _tc_prep_body(x_ref, w_ref, a_ref, h_ref, s_ref):
    xb = x_ref[...]
    nrm = jnp.sqrt(jnp.sum(xb * xb, axis=-1, keepdims=True))
    nrm = jnp.maximum(nrm, MIN_NORM)
    xt = xb * (_artanh(nrm) / nrm)
    one = jnp.ones((xb.shape[0], 1), jnp.float32)
    pad = jnp.zeros((xb.shape[0], DA - D - 1), jnp.float32)
    s1s, s2s = [], []
    for i in range(H):
        h = jnp.dot(xt, w_ref[i], preferred_element_type=jnp.float32)
        h_ref[i] = jnp.concatenate([h, one, pad], axis=-1)
        s1s.append(jnp.sum(h * a_ref[i, :D][None, :], axis=-1, keepdims=True))
        s2s.append(jnp.sum(h * a_ref[i, D:][None, :], axis=-1, keepdims=True))
    s_ref[...] = jnp.concatenate(s1s + s2s, axis=-1)


def _tc_prep(x, W, a):
    return pl.pallas_call(
        _tc_prep_body,
        grid=(N // BN,),
        in_specs=[
            pl.BlockSpec((BN, D), lambda i: (i, 0)),
            pl.BlockSpec((H, D, D), lambda i: (0, 0, 0)),
            pl.BlockSpec((H, 2 * D), lambda i: (0, 0)),
        ],
        out_specs=[
            pl.BlockSpec((H, BN, DA), lambda i: (0, i, 0)),
            pl.BlockSpec((BN, 2 * H), lambda i: (i, 0)),
        ],
        out_shape=[
            jax.ShapeDtypeStruct((H, N, DA), jnp.float32),
            jax.ShapeDtypeStruct((N, 2 * H), jnp.float32),
        ],
    )(x, W, a)


_sc_mesh = plsc.VectorSubcoreMesh(core_axis_name="c", subcore_axis_name="s")


NB = 3
_SC_PARAMS = pltpu.CompilerParams(needs_layout_passes=False,
                                  use_tc_tiling_on_sc=False)


@functools.partial(
    pl.kernel,
    out_type=jax.ShapeDtypeStruct((H * E,), jnp.float32),
    mesh=_sc_mesh,
    compiler_params=_SC_PARAMS,
    scratch_types=[
        pltpu.VMEM((EPT,), jnp.int32),
        pltpu.VMEM((EPT,), jnp.int32),
        pltpu.VMEM((N,), jnp.float32),
        pltpu.VMEM((N,), jnp.float32),
        pltpu.VMEM((N,), jnp.float32),
        pltpu.VMEM((N,), jnp.float32),
        pltpu.VMEM((EPT,), jnp.float32),
        pltpu.VMEM((EPT,), jnp.float32),
    ],
)
def _sc_edge(s_hbm, src_hbm, dst_hbm, e_hbm,
             srcall_v, dstall_v, s1a_v, s2a_v, s1b_v, s2b_v, ea_v, eb_v):
    c = lax.axis_index("c")
    t = lax.axis_index("s")
    ebase = t * EPT
    ha = c
    hb = 2 + c
    pltpu.sync_copy(src_hbm.at[pl.ds(ebase, EPT)], srcall_v)
    pltpu.sync_copy(dst_hbm.at[pl.ds(ebase, EPT)], dstall_v)
    pltpu.sync_copy(s_hbm.at[pl.ds(ha * N, N)], s1a_v)
    pltpu.sync_copy(s_hbm.at[pl.ds((H + ha) * N, N)], s2a_v)
    pltpu.sync_copy(s_hbm.at[pl.ds(hb * N, N)], s1b_v)
    pltpu.sync_copy(s_hbm.at[pl.ds((H + hb) * N, N)], s2b_v)

    def grp(g, carry):
        sl = pl.ds(g * L, L)
        si = srcall_v[sl]
        di = dstall_v[sl]
        za = plsc.load_gather(s1a_v, [si]) + plsc.load_gather(s2a_v, [di])
        zb = plsc.load_gather(s1b_v, [si]) + plsc.load_gather(s2b_v, [di])
        la = jnp.where(za >= 0, za, ALPHA * za)
        lb = jnp.where(zb >= 0, zb, ALPHA * zb)
        ea_v[sl] = jnp.exp(-la)
        eb_v[sl] = jnp.exp(-lb)
        return carry

    lax.fori_loop(0, EPT // L, grp, 0)
    pltpu.sync_copy(ea_v, e_hbm.at[pl.ds(ha * E + ebase, EPT)])
    pltpu.sync_copy(eb_v, e_hbm.at[pl.ds(hb * E + ebase, EPT)])


@functools.partial(
    pl.kernel,
    out_type=jax.ShapeDtypeStruct((H * N, DA), jnp.float32),
    mesh=_sc_mesh,
    compiler_params=_SC_PARAMS,
    scratch_types=(
        [pltpu.VMEM((2 * CH,), jnp.int32)] * NB
        + [pltpu.VMEM((CH,), jnp.int32)] * NB
        + [pltpu.VMEM((CH,), jnp.int32)] * NB
        + [pltpu.VMEM((CH,), jnp.float32)] * NB
        + [pltpu.VMEM((CH, DA), jnp.float32)] * NB
        + [pltpu.VMEM_SHARED((NPAD, DA), jnp.float32)]
        + [pltpu.SemaphoreType.DMA] * (3 * NB)
    ),
)
def _sc_agg(h_hbm, e_hbm, sd_hbm, z_hbm, out_hbm, *refs):
    sdv = refs[0:NB]
    srcsc = refs[NB:2 * NB]
    dstb = refs[2 * NB:3 * NB]
    ev = refs[3 * NB:4 * NB]
    rows = refs[4 * NB:5 * NB]
    hp_sh = refs[5 * NB]
    semi = refs[5 * NB + 1:6 * NB + 1]
    semg = refs[6 * NB + 1:7 * NB + 1]
    sems = refs[7 * NB + 1:8 * NB + 1]
    c = lax.axis_index("c")
    t = lax.axis_index("s")
    rbase = t * RPT
    sdbase = t * (NCHUNK * 2 * CH)
    ebase = t * EPT
    for p in range(2):
        hh = 2 * p + c
        hoff = hh * N
        eoff = hh * E + ebase
        pltpu.sync_copy(z_hbm, hp_sh.at[pl.ds(rbase, RPT)])
        plsc.subcore_barrier()

        def start_idx(kc, cur):
            pltpu.async_copy(sd_hbm.at[pl.ds(sdbase + kc * 2 * CH, 2 * CH)],
                             sdv[cur], semi[cur])
            pltpu.async_copy(e_hbm.at[pl.ds(eoff + kc * CH, CH)],
                             ev[cur], semi[cur])

        def wait_idx(cur):
            pltpu.make_async_copy(sd_hbm.at[pl.ds(0, 2 * CH)], sdv[cur],
                                  semi[cur]).wait()
            pltpu.make_async_copy(e_hbm.at[pl.ds(0, CH)], ev[cur],
                                  semi[cur]).wait()

        def compute(cur):
            for j in range(CH // L):
                sl = pl.ds(j * L, L)
                srcsc[cur][sl] = sdv[cur][pl.ds(j * L, L)]
                dstb[cur][sl] = sdv[cur][pl.ds(CH + j * L, L)] + hoff

        def start_gather(cur):
            pltpu.async_copy(h_hbm.at[dstb[cur]], rows[cur], semg[cur])

        def wait_gather(cur):
            pltpu.make_async_copy(h_hbm.at[pl.ds(0, CH)], rows[cur],
                                  semg[cur]).wait()

        def scale(cur):
            def body(j, carry):
                jb = j * 4
                for dj in range(4):
                    idx = jnp.full((L,), jb + dj, jnp.int32)
                    ee = plsc.load_gather(ev[cur], [idx])
                    for r in range(DA // L):
                        rsl = pl.ds(r * L, L)
                        rows[cur][jb + dj, rsl] = rows[cur][jb + dj, rsl] * ee
                return carry

            lax.fori_loop(0, CH // 4, body, 0)

        def start_scatter(cur):
            pltpu.async_copy(rows[cur], hp_sh.at[srcsc[cur]], sems[cur],
                             add=True)

        def wait_scatter(cur):
            pltpu.make_async_copy(h_hbm.at[pl.ds(0, CH)], rows[cur],
                                  sems[cur]).wait()

        def section(k, cur, drain, prep, prefetch):
            nxt = (cur + 1) % NB
            if drain:
                wait_scatter(nxt)
            if prep:
                wait_idx(nxt)
                compute(nxt)
                start_gather(nxt)
            wait_gather(cur)
            scale(cur)
            start_scatter(cur)
            if prefetch:
                start_idx(k + 3, cur)

        start_idx(0, 0)
        start_idx(1, 1)
        start_idx(2, 2)
        wait_idx(0)
        compute(0)
        start_gather(0)
        section(0, 0, drain=False, prep=True, prefetch=True)
        section(1, 1, drain=False, prep=True, prefetch=True)

        def tri_body(i, carry):
            k0 = 2 + NB * i
            for b in range(NB):
                section(k0 + b, (2 + b) % NB, drain=True, prep=True,
                        prefetch=True)
            return carry

        lax.fori_loop(0, (NCHUNK - 5 - 2) // NB, tri_body, 0)
        section(NCHUNK - 5, (NCHUNK - 5) % NB, True, True, True)
        section(NCHUNK - 4, (NCHUNK - 4) % NB, True, True, True)
        section(NCHUNK - 3, (NCHUNK - 3) % NB, True, True, False)
        section(NCHUNK - 2, (NCHUNK - 2) % NB, True, True, False)
        section(NCHUNK - 1, (NCHUNK - 1) % NB, True, False, False)
        wait_scatter((NCHUNK - 2) % NB)
        wait_scatter((NCHUNK - 1) % NB)
        plsc.subcore_barrier()

        @pl.when(t < NS - 1)
        def _():
            pltpu.sync_copy(hp_sh.at[pl.ds(rbase, RPT)],
                            out_hbm.at[pl.ds(hoff + rbase, RPT)])

        @pl.when(t == NS - 1)
        def _():
            pltpu.sync_copy(hp_sh.at[pl.ds(rbase, RPT_LAST)],
                            out_hbm.at[pl.ds(hoff + rbase, RPT_LAST)])

        plsc.subcore_barrier()


def _tc_finish_body(hp_ref, out_ref):
    hp = hp_ref[...]
    num = hp[:, :, :D]
    den = hp[:, :, D:D + 1]
    g = num / den
    g = jnp.where(g >= 0, g, ACT_SLOPE * g)
    u = jnp.concatenate([g[i] for i in range(H)], axis=-1)
    un = jnp.sqrt(jnp.sum(u * u, axis=-1, keepdims=True))
    un = jnp.maximum(un, MIN_NORM)
    v = jnp.tanh(un) * u / un
    vn = jnp.sqrt(jnp.sum(v * v, axis=-1, keepdims=True))
    vn = jnp.maximum(vn, MIN_NORM)
    maxn = 1.0 - PROJ_EPS
    out_ref[...] = jnp.where(vn > maxn, v / vn * maxn, v)


def _tc_finish(hp):
    return pl.pallas_call(
        _tc_finish_body,
        grid=(N // BN,),
        in_specs=[pl.BlockSpec((H, BN, DA), lambda i: (0, i, 0))],
        out_specs=pl.BlockSpec((BN, H * D), lambda i: (i, 0)),
        out_shape=jax.ShapeDtypeStruct((N, H * D), jnp.float32),
    )(hp)


def kernel(x, adj, W, a):
    src = adj[0]
    dst = adj[1]
    h_aug, s = _tc_prep(x, W, a)
    s_flat = s.T.reshape(-1)
    h_flat = h_aug.reshape(H * N, DA)
    e = _sc_edge(s_flat, src, dst)
    sd = jnp.stack([src.reshape(NS, NCHUNK, CH),
                    dst.reshape(NS, NCHUNK, CH)], axis=2).reshape(-1)
    zeros = jnp.zeros((RPT, DA), jnp.float32)
    hp = _sc_agg(h_flat, e, sd, zeros)
    return _tc_finish(hp.reshape(H, N, DA))
